# baseline (speedup 1.0000x reference)
import functools

import jax
import jax.numpy as jnp
from jax import lax
from jax.experimental import pallas as pl
from jax.experimental.pallas import tpu as pltpu
from jax.experimental.pallas import tpu_sc as plsc

N = 100000
F = 32
E = 1600000

NC = 2
NS = 16
NT = NC * NS
W = 1563
NB = 64
MAGIC = 5367
SHIFT = 23
EPT = E // NT
CH = 1280
NCH = (EPT + CH - 1) // CH
SUB = 128
E_PAD = E + 4096
TRASHBASE = E + 512
ACC_ROWS = 1568
TRASH_ROW = 1564
CSP = 1024

_LANE = None


def _lane():
    return lax.iota(jnp.int32, 16)


_GDN = lax.GatherDimensionNumbers(offset_dims=(), collapsed_slice_dims=(0,),
                                  start_index_map=(0,))


def _rot(v, sh):
    idx = (_lane() + sh) & 15
    return lax.gather(v, idx[:, None], _GDN, (1,),
                      mode=lax.GatherScatterMode.PROMISE_IN_BOUNDS)


def _vsum(v):
    for sh in (1, 2, 4, 8):
        v = v + _rot(v, sh)
    return v[0]


def _bucket(dv):
    return lax.shift_right_logical(dv * MAGIC, SHIFT)


def _hist_body(dst_ref, counts_ref, dst_v, vals_v, pos_v, sem, cnt_smem):
    c = lax.axis_index("c")
    s = lax.axis_index("s")
    tg = c * NS + s
    lane = _lane()
    for b in range(NB + 1):
        cnt_smem[b] = 0
    tile_start = tg * EPT
    tile_end = tile_start + EPT

    def chunk_body(i, _):
        e0p = tile_start + i * CH
        e0 = pl.multiple_of(jnp.minimum(e0p, E - CH), 8)
        pltpu.async_copy(dst_ref.at[pl.ds(e0, CH)], dst_v, sem).wait()

        def grp(g, _):
            o = g * 16
            dv = dst_v[pl.ds(o, 16)]
            b = _bucket(dv)
            phys = e0 + o + lane
            valid = (phys >= e0p) & (phys < tile_end)
            bq = jnp.where(valid, b, NB)
            for u in range(16):
                bu = bq[u]
                cnt_smem[bu] = cnt_smem[bu] + 1
            return ()

        lax.fori_loop(0, CH // 16, grp, ())
        return ()

    lax.fori_loop(0, NCH, chunk_body, ())

    for gb in range(4):
        vv = jnp.zeros((16,), jnp.int32)
        for u in range(16):
            vv = jnp.where(lane == u, cnt_smem[gb * 16 + u], vv)
        vals_v[pl.ds(gb * 16, 16)] = vv
        pos_v[0, pl.ds(gb * 16, 16)] = (gb * 16 + lane) * NT + tg
    pltpu.async_copy(vals_v, counts_ref.at[pos_v.at[0]], sem).wait()


_hist = functools.partial(
    pl.kernel,
    out_type=jax.ShapeDtypeStruct((NB * NT,), jnp.int32),
    mesh=plsc.VectorSubcoreMesh(core_axis_name="c", subcore_axis_name="s",
                                num_cores=NC, num_subcores=NS),
    compiler_params=pltpu.CompilerParams(use_tc_tiling_on_sc=False),
    scratch_types=[
        pltpu.VMEM((CH,), jnp.int32),
        pltpu.VMEM((NB,), jnp.int32),
        pltpu.VMEM((1, NB), jnp.int32),
        pltpu.SemaphoreType.DMA,
        pltpu.SMEM((NB + 1,), jnp.int32),
    ],
)(_hist_body)


def _scat_body(src_ref, dst_ref, w_ref, counts_ref,
               psrc_ref, plidx_ref, pw_ref, aux_ref,
               cnts_v, src_v, dst_v, w_v, lidx_v, posbuf_v, auxrow_v,
               sem, sem2, off_smem):
    c = lax.axis_index("c")
    s = lax.axis_index("s")
    tg = c * NS + s
    lane = _lane()
    pltpu.async_copy(counts_ref, cnts_v, sem).wait()

    base = jnp.int32(0)
    starts = []
    tots = []
    for b in range(NB):
        v0 = cnts_v[pl.ds(b * NT, 16)]
        v1 = cnts_v[pl.ds(b * NT + 16, 16)]
        tot = _vsum(v0) + _vsum(v1)
        pre = (_vsum(jnp.where(lane < tg, v0, 0))
               + _vsum(jnp.where(lane < tg - 16, v1, 0)))
        off_smem[b] = base + pre
        starts.append(base)
        tots.append(tot)
        base = (base + tot + 7) & (-8)
    off_smem[NB] = TRASHBASE

    b0 = 2 * tg
    s0 = jnp.int32(0)
    c0 = jnp.int32(0)
    s1 = jnp.int32(0)
    c1 = jnp.int32(0)
    for b in range(NB):
        m0 = b0 == b
        m1 = (b0 + 1) == b
        s0 = jnp.where(m0, starts[b], s0)
        c0 = jnp.where(m0, tots[b], c0)
        s1 = jnp.where(m1, starts[b], s1)
        c1 = jnp.where(m1, tots[b], c1)
    row = jnp.zeros((16,), jnp.int32)
    row = jnp.where(lane == 0, s0, row)
    row = jnp.where(lane == 1, c0, row)
    row = jnp.where(lane == 2, s1, row)
    row = jnp.where(lane == 3, c1, row)
    auxrow_v[pl.ds(0, 16)] = row
    pltpu.async_copy(auxrow_v, aux_ref.at[tg], sem).wait()

    tile_start = tg * EPT
    tile_end = tile_start + EPT
    onehot = [None] * 16
    for u in range(16):
        onehot[u] = lane == u

    def chunk_body(i, _):
        e0p = tile_start + i * CH
        e0 = pl.multiple_of(jnp.minimum(e0p, E - CH), 8)
        d1 = pltpu.async_copy(src_ref.at[pl.ds(e0, CH)], src_v, sem2)
        d2 = pltpu.async_copy(dst_ref.at[pl.ds(e0, CH)], dst_v, sem2)
        d3 = pltpu.async_copy(w_ref.at[pl.ds(e0, CH)], w_v, sem2)
        d1.wait()
        d2.wait()
        d3.wait()

        def sub(j, _):
            for gg in range(8):
                o = j * SUB + gg * 16
                dv = dst_v[pl.ds(o, 16)]
                b = _bucket(dv)
                lv = dv - b * W
                phys = e0 + o + lane
                valid = (phys >= e0p) & (phys < tile_end)
                bq = jnp.where(valid, b, NB)
                lidx_v[pl.ds(o, 16)] = lv
                posv = jnp.zeros((16,), jnp.int32)
                for u in range(16):
                    bu = bq[u]
                    p = off_smem[bu]
                    off_smem[bu] = p + 1
                    posv = jnp.where(onehot[u], p, posv)
                posbuf_v[j, pl.ds(gg * 16, 16)] = posv
            return ()

        lax.fori_loop(0, CH // SUB, sub, ())
        ds_ = []
        for j in range(0):
            jo = j * SUB
            ds_.append(pltpu.async_copy(
                src_v.at[pl.ds(jo, SUB)], psrc_ref.at[posbuf_v.at[j]], sem))
            ds_.append(pltpu.async_copy(
                lidx_v.at[pl.ds(jo, SUB)], plidx_ref.at[posbuf_v.at[j]], sem))
            ds_.append(pltpu.async_copy(
                w_v.at[pl.ds(jo, SUB)], pw_ref.at[posbuf_v.at[j]], sem))
        for d in ds_:
            d.wait()
        return ()

    lax.fori_loop(0, NCH, chunk_body, ())


_scat = functools.partial(
    pl.kernel,
    out_type=(jax.ShapeDtypeStruct((E_PAD,), jnp.int32),
              jax.ShapeDtypeStruct((E_PAD,), jnp.int32),
              jax.ShapeDtypeStruct((E_PAD,), jnp.float32),
              jax.ShapeDtypeStruct((NT, 16), jnp.int32)),
    mesh=plsc.VectorSubcoreMesh(core_axis_name="c", subcore_axis_name="s",
                                num_cores=NC, num_subcores=NS),
    compiler_params=pltpu.CompilerParams(use_tc_tiling_on_sc=False),
    scratch_types=[
        pltpu.VMEM((NB * NT,), jnp.int32),
        pltpu.VMEM((CH,), jnp.int32),
        pltpu.VMEM((CH,), jnp.int32),
        pltpu.VMEM((CH,), jnp.float32),
        pltpu.VMEM((CH,), jnp.int32),
        pltpu.VMEM((CH // SUB, SUB), jnp.int32),
        pltpu.VMEM((16,), jnp.int32),
        pltpu.SemaphoreType.DMA,
        pltpu.SemaphoreType.DMA,
        pltpu.SMEM((NB + 1,), jnp.int32),
    ],
)(_scat_body)


def _spmm_body(psrc_ref, plidx_ref, pw_ref, aux_ref, sup_ref, out_ref,
               aux_v, src_v, lidx_v, w_v, rows_v, acc_v, semg, semi):
    c = lax.axis_index("c")
    s = lax.axis_index("s")
    tg = c * NS + s
    lane = _lane()
    pltpu.async_copy(aux_ref.at[tg], aux_v, semi).wait()
    row = aux_v[pl.ds(0, 16)]
    zero16 = jnp.zeros((16,), jnp.float32)

    for bsel in range(2):
        st = row[2 * bsel]
        cn = row[2 * bsel + 1]

        def zr(r, _):
            for u in range(8):
                acc_v[r * 8 + u, pl.ds(0, 16)] = zero16
                acc_v[r * 8 + u, pl.ds(16, 16)] = zero16
            return ()

        lax.fori_loop(0, ACC_ROWS // 8, zr, ())

        nch = lax.shift_right_logical(cn + (CSP - 1), 10)

        def chunk_body(i, _):
            e0 = pl.multiple_of(st + i * CSP, 8)
            d1 = pltpu.async_copy(psrc_ref.at[pl.ds(e0, CSP)], src_v, semi)
            d2 = pltpu.async_copy(plidx_ref.at[pl.ds(e0, CSP)], lidx_v, semi)
            d3 = pltpu.async_copy(pw_ref.at[pl.ds(e0, CSP)], w_v, semi)
            d1.wait()

            def clamp(g, _):
                o = g * 16
                v = src_v[pl.ds(o, 16)]
                src_v[pl.ds(o, 16)] = jnp.minimum(jnp.maximum(v, 0), N - 1)
                return ()

            lax.fori_loop(0, CSP // 16, clamp, ())
            gathers = []
            for j in range(CSP // SUB):
                jo = j * SUB
                gathers.append(pltpu.async_copy(
                    sup_ref.at[src_v.at[pl.ds(jo, SUB)]],
                    rows_v.at[pl.ds(jo, SUB)], semg))
            d2.wait()
            d3.wait()
            for j in range(CSP // SUB):
                gathers[j].wait()

                def mgrp(gg, _, j=j):
                    o = j * SUB + gg * 16
                    lv = lidx_v[pl.ds(o, 16)]
                    eidx = i * CSP + o + lane
                    lv = jnp.minimum(jnp.maximum(lv, 0), TRASH_ROW)
                    lv = jnp.where(eidx >= cn, TRASH_ROW, lv)
                    wv = w_v[pl.ds(o, 16)]
                    for u in range(16):
                        er = o + u
                        wsc = wv[u]
                        lsc = lv[u]
                        r0 = rows_v[er, pl.ds(0, 16)]
                        r1 = rows_v[er, pl.ds(16, 16)]
                        plsc.addupdate(acc_v.at[lsc, pl.ds(0, 16)], r0 * wsc)
                        plsc.addupdate(acc_v.at[lsc, pl.ds(16, 16)], r1 * wsc)
                    return ()

                lax.fori_loop(0, SUB // 16, mgrp, ())
            return ()

        lax.fori_loop(0, nch, chunk_body, ())

        bglob = 2 * tg + bsel
        obase = bglob * W

        @pl.when(bglob == NB - 1)
        def _():
            pltpu.sync_copy(acc_v.at[pl.ds(0, N - (NB - 1) * W)],
                            out_ref.at[pl.ds(obase, N - (NB - 1) * W)])

        @pl.when(bglob != NB - 1)
        def _():
            pltpu.sync_copy(acc_v.at[pl.ds(0, W)],
                            out_ref.at[pl.ds(obase, W)])


_spmm = functools.partial(
    pl.kernel,
    out_type=jax.ShapeDtypeStruct((N, F), jnp.float32),
    mesh=plsc.VectorSubcoreMesh(core_axis_name="c", subcore_axis_name="s",
                                num_cores=NC, num_subcores=NS),
    compiler_params=pltpu.CompilerParams(use_tc_tiling_on_sc=False),
    scratch_types=[
        pltpu.VMEM((16,), jnp.int32),
        pltpu.VMEM((CSP,), jnp.int32),
        pltpu.VMEM((CSP,), jnp.int32),
        pltpu.VMEM((CSP,), jnp.float32),
        pltpu.VMEM((CSP, F), jnp.float32),
        pltpu.VMEM((ACC_ROWS, F), jnp.float32),
        pltpu.SemaphoreType.DMA,
        pltpu.SemaphoreType.DMA,
    ],
)(_spmm_body)


R = 2000
GRID = N // R


def _norm_mm_body(a_ref, w_ref, x_ref, s_ref):
    x = jnp.maximum(a_ref[...], 0.0)
    n = jnp.sqrt(jnp.sum(x * x, axis=1, keepdims=True))
    x = x / jnp.maximum(n, 1e-12)
    x_ref[...] = x
    s_ref[...] = jnp.dot(x, w_ref[...], preferred_element_type=jnp.float32)


def _norm_mm(a, w):
    return pl.pallas_call(
        _norm_mm_body,
        grid=(GRID,),
        in_specs=[pl.BlockSpec((R, F), lambda i: (i, 0)),
                  pl.BlockSpec((F, F), lambda i: (0, 0))],
        out_specs=[pl.BlockSpec((R, F), lambda i: (i, 0)),
                   pl.BlockSpec((R, F), lambda i: (i, 0))],
        out_shape=[jax.ShapeDtypeStruct((N, F), jnp.float32),
                   jax.ShapeDtypeStruct((N, F), jnp.float32)],
    )(a, w)


def _mlp_body(x1_ref, x2_ref, x3_ref, x4_ref, x5_ref, a6_ref,
              l1w_ref, l1b_ref, l2w_ref, l2b_ref, l3w_ref, l3b_ref,
              prev_ref, out_ref):
    xs = [x1_ref[...], x2_ref[...], x3_ref[...], x4_ref[...], x5_ref[...]]
    x6 = jnp.maximum(a6_ref[...], 0.0)
    x7 = xs[0] + xs[1] + xs[2] + xs[3] + xs[4] + x6
    l1w, l1b = l1w_ref[...], l1b_ref[...]
    l2w, l2b = l2w_ref[...], l2b_ref[...]
    l3w, l3b = l3w_ref[...], l3b_ref[...]
    total = jnp.zeros((R, 1), jnp.float32)
    for xi in (*xs, x6, x7):
        h = jnp.maximum(jnp.dot(xi, l1w, preferred_element_type=jnp.float32) + l1b, 0.0)
        h = jnp.maximum(jnp.dot(h, l2w, preferred_element_type=jnp.float32) + l2b, 0.0)
        total = total + jnp.dot(h, l3w, preferred_element_type=jnp.float32) + l3b
    out_ref[...] = prev_ref[...] * (total * (1.0 / 7.0))


def _mlp(x1, x2, x3, x4, x5, a6, l1w, l1b, l2w, l2b, l3w, l3b, prev):
    H = 2 * F
    xspec = pl.BlockSpec((R, F), lambda i: (i, 0))
    return pl.pallas_call(
        _mlp_body,
        grid=(GRID,),
        in_specs=[xspec] * 6 + [
            pl.BlockSpec((F, H), lambda i: (0, 0)),
            pl.BlockSpec((1, H), lambda i: (0, 0)),
            pl.BlockSpec((H, H), lambda i: (0, 0)),
            pl.BlockSpec((1, H), lambda i: (0, 0)),
            pl.BlockSpec((H, 1), lambda i: (0, 0)),
            pl.BlockSpec((1, 1), lambda i: (0, 0)),
            pl.BlockSpec((R, 1), lambda i: (i, 0)),
        ],
        out_specs=pl.BlockSpec((R, 1), lambda i: (i, 0)),
        out_shape=jax.ShapeDtypeStruct((N, 1), jnp.float32),
    )(x1, x2, x3, x4, x5, a6, l1w, l1b.reshape(1, H), l2w,
      l2b.reshape(1, H), l3w, l3b.reshape(1, 1), prev)


def kernel(edge_index1, edge_weight1, edge_index2, edge_weight2,
           W1, W2, W3, W4, W5, W6, l1w, l1b, l2w, l2b, l3w, l3b):
    ones = jnp.ones((N, 1), jnp.float32)

    def branch(ei, ew, prev):
        src = ei[1].astype(jnp.int32)
        dst = ei[0].astype(jnp.int32)
        counts = _hist(dst)
        psrc, plidx, pw, aux = _scat(src, dst, ew, counts)
        a1 = _spmm(psrc, plidx, pw, aux, W1)
        x1, s2 = _norm_mm(a1, W2)
        a2 = _spmm(psrc, plidx, pw, aux, s2)
        x2, s3 = _norm_mm(a2, W3)
        a3 = _spmm(psrc, plidx, pw, aux, s3)
        x3, s4 = _norm_mm(a3, W4)
        a4 = _spmm(psrc, plidx, pw, aux, s4)
        x4, s5 = _norm_mm(a4, W5)
        a5 = _spmm(psrc, plidx, pw, aux, s5)
        x5, s6 = _norm_mm(a5, W6)
        a6 = _spmm(psrc, plidx, pw, aux, s6)
        return _mlp(x1, x2, x3, x4, x5, a6,
                    l1w, l1b, l2w, l2b, l3w, l3b, prev)

    score1 = branch(edge_index1, edge_weight1, ones)
    return branch(edge_index2, edge_weight2, score1)


# speedup vs baseline: 11.1693x; 11.1693x over previous
import functools

import jax
import jax.numpy as jnp
from jax import lax
from jax.experimental import pallas as pl
from jax.experimental.pallas import tpu as pltpu
from jax.experimental.pallas import tpu_sc as plsc

N = 100000
F = 32
E = 1600000

NC = 2
NS = 16
NT = NC * NS
W = 1563
NB = 64
MAGIC = 5367
SHIFT = 23
EPT = E // NT
CH = 1280
NCHUNKS = E // CH
SUB = 128
EROWS = E // 128
E_PAD = E + 4096
TRASHBASE = E + 512
ACC_ROWS = 1568
TRASH_ROW = 1564
CSP = 1024

_LANE = None


def _lane():
    return lax.iota(jnp.int32, 16)


_GDN = lax.GatherDimensionNumbers(offset_dims=(), collapsed_slice_dims=(0,),
                                  start_index_map=(0,))


def _rot(v, sh):
    idx = (_lane() + sh) & 15
    return lax.gather(v, idx[:, None], _GDN, (1,),
                      mode=lax.GatherScatterMode.PROMISE_IN_BOUNDS)


def _vsum(v):
    for sh in (1, 2, 4, 8):
        v = v + _rot(v, sh)
    return v[0]


def _bucket(dv):
    return lax.shift_right_logical(dv * MAGIC, SHIFT)


def _hist_body(dst_ref, counts_ref, dst_v, vals_v, pos_v, sem, cnt_smem):
    c = lax.axis_index("c")
    s = lax.axis_index("s")
    tg = c * NS + s
    lane = _lane()
    for b in range(NB + 1):
        cnt_smem[b] = 0

    def chunk_body(i, _):
        r0 = (tg + i * NT) * (CH // SUB)
        pltpu.async_copy(dst_ref.at[pl.ds(r0, CH // SUB)], dst_v, sem).wait()

        def sub(j, _):
            for gg in range(8):
                b = _bucket(dst_v[j, pl.ds(gg * 16, 16)])
                bs = [b[u] for u in range(16)]
                for bu in bs:
                    cnt_smem[bu] = cnt_smem[bu] + 1
            return ()

        lax.fori_loop(0, CH // SUB, sub, ())
        return ()

    n_t = jnp.where(tg < NCHUNKS - (NCHUNKS // NT) * NT,
                    NCHUNKS // NT + 1, NCHUNKS // NT)
    lax.fori_loop(0, n_t, chunk_body, ())

    for gb in range(4):
        vv = jnp.zeros((16,), jnp.int32)
        for u in range(16):
            vv = jnp.where(lane == u, cnt_smem[gb * 16 + u], vv)
        vals_v[pl.ds(gb * 16, 16)] = vv
        pos_v[0, pl.ds(gb * 16, 16)] = (gb * 16 + lane) * NT + tg
    pltpu.async_copy(vals_v, counts_ref.at[pos_v.at[0]], sem).wait()


_hist = functools.partial(
    pl.kernel,
    out_type=jax.ShapeDtypeStruct((NB * NT,), jnp.int32),
    mesh=plsc.VectorSubcoreMesh(core_axis_name="c", subcore_axis_name="s",
                                num_cores=NC, num_subcores=NS),
    compiler_params=pltpu.CompilerParams(use_tc_tiling_on_sc=False),
    scratch_types=[
        pltpu.VMEM((CH // SUB, SUB), jnp.int32),
        pltpu.VMEM((NB,), jnp.int32),
        pltpu.VMEM((1, NB), jnp.int32),
        pltpu.SemaphoreType.DMA,
        pltpu.SMEM((NB + 1,), jnp.int32),
    ],
)(_hist_body)


def _scat_body(src_ref, dst_ref, w_ref, counts_ref,
               psrc_ref, plidx_ref, pw_ref, aux_ref,
               cnts_v, src_v, dst_v, w_v, lidx_v, posflat_v, auxrow_v,
               sem, sem2, off_smem):
    c = lax.axis_index("c")
    s = lax.axis_index("s")
    tg = c * NS + s
    lane = _lane()
    pltpu.async_copy(counts_ref, cnts_v, sem).wait()

    base = jnp.int32(0)
    starts = []
    tots = []
    for b in range(NB):
        v0 = cnts_v[pl.ds(b * NT, 16)]
        v1 = cnts_v[pl.ds(b * NT + 16, 16)]
        tot = _vsum(v0) + _vsum(v1)
        pre = (_vsum(jnp.where(lane < tg, v0, 0))
               + _vsum(jnp.where(lane < tg - 16, v1, 0)))
        off_smem[b] = base + pre
        starts.append(base)
        tots.append(tot)
        base = (base + tot + 7) & (-8)
    off_smem[NB] = TRASHBASE

    b0 = 2 * tg
    s0 = jnp.int32(0)
    c0 = jnp.int32(0)
    s1 = jnp.int32(0)
    c1 = jnp.int32(0)
    for b in range(NB):
        m0 = b0 == b
        m1 = (b0 + 1) == b
        s0 = jnp.where(m0, starts[b], s0)
        c0 = jnp.where(m0, tots[b], c0)
        s1 = jnp.where(m1, starts[b], s1)
        c1 = jnp.where(m1, tots[b], c1)
    row = jnp.zeros((16,), jnp.int32)
    row = jnp.where(lane == 0, s0, row)
    row = jnp.where(lane == 1, c0, row)
    row = jnp.where(lane == 2, s1, row)
    row = jnp.where(lane == 3, c1, row)
    auxrow_v[pl.ds(0, 16)] = row
    pltpu.async_copy(auxrow_v, aux_ref.at[tg], sem).wait()

    onehot = [None] * 16
    for u in range(16):
        onehot[u] = lane == u

    def chunk_body(i, _):
        e0 = pl.multiple_of((tg + i * NT) * CH, SUB)
        d1 = pltpu.async_copy(src_ref.at[pl.ds(e0, CH)], src_v, sem2)
        d2 = pltpu.async_copy(dst_ref.at[pl.ds(e0, CH)], dst_v, sem2)
        d3 = pltpu.async_copy(w_ref.at[pl.ds(e0, CH)], w_v, sem2)
        d1.wait()
        d2.wait()
        d3.wait()

        def sub(j, _):
            for gg in range(8):
                o = j * SUB + gg * 16
                dv = dst_v[pl.ds(o, 16)]
                b = _bucket(dv)
                lv = dv - b * W
                lidx_v[pl.ds(o, 16)] = lv
                bs = [b[u] for u in range(16)]
                posv = lane * 0
                for u in range(16):
                    bu = bs[u]
                    p = off_smem[bu]
                    off_smem[bu] = p + 1
                    posv = jnp.where(onehot[u], p, posv)
                posflat_v[pl.ds(o, 16)] = posv
            return ()

        lax.fori_loop(0, CH // SUB, sub, ())
        d4 = pltpu.async_copy(src_v, psrc_ref.at[posflat_v], sem)
        d5 = pltpu.async_copy(lidx_v, plidx_ref.at[posflat_v], sem)
        d6 = pltpu.async_copy(w_v, pw_ref.at[posflat_v], sem)
        d4.wait()
        d5.wait()
        d6.wait()
        return ()

    n_t = jnp.where(tg < NCHUNKS - (NCHUNKS // NT) * NT,
                    NCHUNKS // NT + 1, NCHUNKS // NT)
    lax.fori_loop(0, n_t, chunk_body, ())


_scat = functools.partial(
    pl.kernel,
    out_type=(jax.ShapeDtypeStruct((E_PAD,), jnp.int32),
              jax.ShapeDtypeStruct((E_PAD,), jnp.int32),
              jax.ShapeDtypeStruct((E_PAD,), jnp.float32),
              jax.ShapeDtypeStruct((NT, 16), jnp.int32)),
    mesh=plsc.VectorSubcoreMesh(core_axis_name="c", subcore_axis_name="s",
                                num_cores=NC, num_subcores=NS),
    compiler_params=pltpu.CompilerParams(use_tc_tiling_on_sc=False),
    scratch_types=[
        pltpu.VMEM((NB * NT,), jnp.int32),
        pltpu.VMEM((CH,), jnp.int32),
        pltpu.VMEM((CH,), jnp.int32),
        pltpu.VMEM((CH,), jnp.float32),
        pltpu.VMEM((CH,), jnp.int32),
        pltpu.VMEM((CH,), jnp.int32),
        pltpu.VMEM((16,), jnp.int32),
        pltpu.SemaphoreType.DMA,
        pltpu.SemaphoreType.DMA,
        pltpu.SMEM((NB + 1,), jnp.int32),
    ],
)(_scat_body)


def _spmm_body(psrc_ref, plidx_ref, pw_ref, aux_ref, sup_ref, out_ref,
               aux_v, src_v, lidx_v, w_v, rows_v, acc_v, semg, semi):
    c = lax.axis_index("c")
    s = lax.axis_index("s")
    tg = c * NS + s
    lane = _lane()
    pltpu.async_copy(aux_ref.at[tg], aux_v, semi).wait()
    row = aux_v[pl.ds(0, 16)]
    zero16 = jnp.zeros((16,), jnp.float32)

    for bsel in range(2):
        st = row[2 * bsel]
        cn = row[2 * bsel + 1]

        def zr(r, _):
            for u in range(8):
                acc_v[r * 8 + u, pl.ds(0, 16)] = zero16
                acc_v[r * 8 + u, pl.ds(16, 16)] = zero16
            return ()

        lax.fori_loop(0, ACC_ROWS // 8, zr, ())

        nch = lax.shift_right_logical(cn + (CSP - 1), 10)

        def chunk_body(i, _):
            e0 = pl.multiple_of(st + i * CSP, 8)
            d1 = pltpu.async_copy(psrc_ref.at[pl.ds(e0, CSP)], src_v, semi)
            d2 = pltpu.async_copy(plidx_ref.at[pl.ds(e0, CSP)], lidx_v, semi)
            d3 = pltpu.async_copy(pw_ref.at[pl.ds(e0, CSP)], w_v, semi)
            d1.wait()

            def clamp(g, _):
                o = g * 16
                v = src_v[pl.ds(o, 16)]
                src_v[pl.ds(o, 16)] = jnp.minimum(jnp.maximum(v, 0), N - 1)
                return ()

            lax.fori_loop(0, CSP // 16, clamp, ())
            gathers = []
            for j in range(CSP // SUB):
                jo = j * SUB
                gathers.append(pltpu.async_copy(
                    sup_ref.at[src_v.at[pl.ds(jo, SUB)]],
                    rows_v.at[pl.ds(jo, SUB)], semg))
            d2.wait()
            d3.wait()
            for j in range(CSP // SUB):
                gathers[j].wait()

                def mgrp(gg, _, j=j):
                    o = j * SUB + gg * 16
                    lv = lidx_v[pl.ds(o, 16)]
                    eidx = i * CSP + o + lane
                    lv = jnp.minimum(jnp.maximum(lv, 0), TRASH_ROW)
                    lv = jnp.where(eidx >= cn, TRASH_ROW, lv)
                    wv = w_v[pl.ds(o, 16)]
                    ls = [lv[u] for u in range(16)]
                    ws = [wv[u] for u in range(16)]
                    for u in range(16):
                        er = o + u
                        r0 = rows_v[er, pl.ds(0, 16)]
                        r1 = rows_v[er, pl.ds(16, 16)]
                        plsc.addupdate(acc_v.at[ls[u], pl.ds(0, 16)], r0 * ws[u])
                        plsc.addupdate(acc_v.at[ls[u], pl.ds(16, 16)], r1 * ws[u])
                    return ()

                lax.fori_loop(0, SUB // 16, mgrp, ())
            return ()

        lax.fori_loop(0, nch, chunk_body, ())

        bglob = 2 * tg + bsel
        obase = bglob * W

        @pl.when(bglob == NB - 1)
        def _():
            pltpu.sync_copy(acc_v.at[pl.ds(0, N - (NB - 1) * W)],
                            out_ref.at[pl.ds(obase, N - (NB - 1) * W)])

        @pl.when(bglob != NB - 1)
        def _():
            pltpu.sync_copy(acc_v.at[pl.ds(0, W)],
                            out_ref.at[pl.ds(obase, W)])


_spmm = functools.partial(
    pl.kernel,
    out_type=jax.ShapeDtypeStruct((N, F), jnp.float32),
    mesh=plsc.VectorSubcoreMesh(core_axis_name="c", subcore_axis_name="s",
                                num_cores=NC, num_subcores=NS),
    compiler_params=pltpu.CompilerParams(use_tc_tiling_on_sc=False),
    scratch_types=[
        pltpu.VMEM((16,), jnp.int32),
        pltpu.VMEM((CSP,), jnp.int32),
        pltpu.VMEM((CSP,), jnp.int32),
        pltpu.VMEM((CSP,), jnp.float32),
        pltpu.VMEM((CSP, F), jnp.float32),
        pltpu.VMEM((ACC_ROWS, F), jnp.float32),
        pltpu.SemaphoreType.DMA,
        pltpu.SemaphoreType.DMA,
    ],
)(_spmm_body)


R = 2000
GRID = N // R


def _norm_mm_body(a_ref, w_ref, x_ref, s_ref):
    x = jnp.maximum(a_ref[...], 0.0)
    n = jnp.sqrt(jnp.sum(x * x, axis=1, keepdims=True))
    x = x / jnp.maximum(n, 1e-12)
    x_ref[...] = x
    s_ref[...] = jnp.dot(x, w_ref[...], preferred_element_type=jnp.float32)


def _norm_mm(a, w):
    return pl.pallas_call(
        _norm_mm_body,
        grid=(GRID,),
        in_specs=[pl.BlockSpec((R, F), lambda i: (i, 0)),
                  pl.BlockSpec((F, F), lambda i: (0, 0))],
        out_specs=[pl.BlockSpec((R, F), lambda i: (i, 0)),
                   pl.BlockSpec((R, F), lambda i: (i, 0))],
        out_shape=[jax.ShapeDtypeStruct((N, F), jnp.float32),
                   jax.ShapeDtypeStruct((N, F), jnp.float32)],
    )(a, w)


def _mlp_body(x1_ref, x2_ref, x3_ref, x4_ref, x5_ref, a6_ref,
              l1w_ref, l1b_ref, l2w_ref, l2b_ref, l3w_ref, l3b_ref,
              prev_ref, out_ref):
    xs = [x1_ref[...], x2_ref[...], x3_ref[...], x4_ref[...], x5_ref[...]]
    x6 = jnp.maximum(a6_ref[...], 0.0)
    x7 = xs[0] + xs[1] + xs[2] + xs[3] + xs[4] + x6
    l1w, l1b = l1w_ref[...], l1b_ref[...]
    l2w, l2b = l2w_ref[...], l2b_ref[...]
    l3w, l3b = l3w_ref[...], l3b_ref[...]
    total = jnp.zeros((R, 1), jnp.float32)
    for xi in (*xs, x6, x7):
        h = jnp.maximum(jnp.dot(xi, l1w, preferred_element_type=jnp.float32) + l1b, 0.0)
        h = jnp.maximum(jnp.dot(h, l2w, preferred_element_type=jnp.float32) + l2b, 0.0)
        total = total + jnp.dot(h, l3w, preferred_element_type=jnp.float32) + l3b
    out_ref[...] = prev_ref[...] * (total * (1.0 / 7.0))


def _mlp(x1, x2, x3, x4, x5, a6, l1w, l1b, l2w, l2b, l3w, l3b, prev):
    H = 2 * F
    xspec = pl.BlockSpec((R, F), lambda i: (i, 0))
    return pl.pallas_call(
        _mlp_body,
        grid=(GRID,),
        in_specs=[xspec] * 6 + [
            pl.BlockSpec((F, H), lambda i: (0, 0)),
            pl.BlockSpec((1, H), lambda i: (0, 0)),
            pl.BlockSpec((H, H), lambda i: (0, 0)),
            pl.BlockSpec((1, H), lambda i: (0, 0)),
            pl.BlockSpec((H, 1), lambda i: (0, 0)),
            pl.BlockSpec((1, 1), lambda i: (0, 0)),
            pl.BlockSpec((R, 1), lambda i: (i, 0)),
        ],
        out_specs=pl.BlockSpec((R, 1), lambda i: (i, 0)),
        out_shape=jax.ShapeDtypeStruct((N, 1), jnp.float32),
    )(x1, x2, x3, x4, x5, a6, l1w, l1b.reshape(1, H), l2w,
      l2b.reshape(1, H), l3w, l3b.reshape(1, 1), prev)


def kernel(edge_index1, edge_weight1, edge_index2, edge_weight2,
           W1, W2, W3, W4, W5, W6, l1w, l1b, l2w, l2b, l3w, l3b):
    ones = jnp.ones((N, 1), jnp.float32)

    def branch(ei, ew, prev):
        src = ei[1].astype(jnp.int32).reshape(EROWS, SUB)
        dst = ei[0].astype(jnp.int32).reshape(EROWS, SUB)
        w2d = ew.reshape(EROWS, SUB)
        counts = _hist(dst)
        psrc, plidx, pw, aux = _scat(ei[1].astype(jnp.int32),
                                     ei[0].astype(jnp.int32), ew, counts)
        a1 = _spmm(psrc, plidx, pw, aux, W1)
        x1, s2 = _norm_mm(a1, W2)
        a2 = _spmm(psrc, plidx, pw, aux, s2)
        x2, s3 = _norm_mm(a2, W3)
        a3 = _spmm(psrc, plidx, pw, aux, s3)
        x3, s4 = _norm_mm(a3, W4)
        a4 = _spmm(psrc, plidx, pw, aux, s4)
        x4, s5 = _norm_mm(a4, W5)
        a5 = _spmm(psrc, plidx, pw, aux, s5)
        x5, s6 = _norm_mm(a5, W6)
        a6 = _spmm(psrc, plidx, pw, aux, s6)
        return _mlp(x1, x2, x3, x4, x5, a6,
                    l1w, l1b, l2w, l2b, l3w, l3b, prev)

    score1 = branch(edge_index1, edge_weight1, ones)
    return branch(edge_index2, edge_weight2, score1)
